# baseline (speedup 1.0000x reference)
import jax
import jax.numpy as jnp
from jax import lax
from jax.experimental import pallas as pl
from jax.experimental.pallas import tpu as pltpu
from jax.experimental.pallas import tpu_sc as plsc

_L = 16
_CTX = 2048
_LAYERS = 2
_BATCH = 8


def _sc_kv_fetch(si_hbm, li_hbm, ks_hbm, vs_hbm, kc_hbm, vc_hbm,
                 ko_hbm, vo_hbm, pvm, slab, sem_p, sem_s):
    wid = lax.axis_index("s")
    b = jnp.where(wid < _BATCH, wid, wid - _BATCH)
    is_k = wid < _BATCH

    @pl.when(is_k)
    def _():
        pltpu.async_copy(ks_hbm.at[b, 0], slab, sem_s)

    @pl.when(jnp.logical_not(is_k))
    def _():
        pltpu.async_copy(vs_hbm.at[b, 0], slab, sem_s)

    c1 = pltpu.async_copy(si_hbm, pvm.at[pl.ds(0, 2)], sem_p)
    c2 = pltpu.async_copy(li_hbm, pvm.at[pl.ds(8, 1)], sem_p)
    c1.wait()
    c2.wait()
    v = pvm[pl.ds(0, _L)]
    begin_raw = v[0]
    end_raw = v[1]
    li_raw = v[8]
    li = jnp.clip(li_raw, 0, _LAYERS - 1)
    begin = jnp.clip(begin_raw, 0, _CTX - 1)
    p = jnp.clip(end_raw - 1, 0, _CTX - 1)
    hit = p == begin

    pltpu.make_async_copy(ks_hbm.at[0, 0], slab, sem_s).wait()

    @pl.when(is_k & jnp.logical_not(hit))
    def _():
        pltpu.sync_copy(kc_hbm.at[li, b, p], slab)

    @pl.when(jnp.logical_not(is_k) & jnp.logical_not(hit))
    def _():
        pltpu.sync_copy(vc_hbm.at[li, b, p], slab)

    @pl.when(is_k)
    def _():
        pltpu.sync_copy(slab, ko_hbm.at[b, 0])

    @pl.when(jnp.logical_not(is_k))
    def _():
        pltpu.sync_copy(slab, vo_hbm.at[b, 0])


def kernel(k_state, v_state, layer_idx, slice_indices, k_cache, v_cache):
    si = slice_indices.astype(jnp.int32)
    li = jnp.asarray(layer_idx, jnp.int32).reshape(1)

    out_sds = jax.ShapeDtypeStruct(k_state.shape, k_state.dtype)
    mesh = plsc.VectorSubcoreMesh(
        core_axis_name="c", subcore_axis_name="s", num_cores=1)
    run = pl.kernel(
        _sc_kv_fetch,
        mesh=mesh,
        out_type=(out_sds, out_sds),
        scratch_types=[
            pltpu.VMEM((_L,), jnp.int32),
            pltpu.VMEM((_BATCH, 128), jnp.float32),
            pltpu.SemaphoreType.DMA,
            pltpu.SemaphoreType.DMA,
        ],
    )
    k_out, v_out = run(si, li, k_state, v_state, k_cache, v_cache)
    return (k_out, v_out)


# speedup vs baseline: 1.0036x; 1.0036x over previous
import jax
import jax.numpy as jnp
from jax import lax
from jax.experimental import pallas as pl
from jax.experimental.pallas import tpu as pltpu
from jax.experimental.pallas import tpu_sc as plsc

_L = 16
_CTX = 2048
_LAYERS = 2
_BATCH = 8


def _sc_kv_fetch(si_hbm, li_hbm, ks_hbm, vs_hbm, kc_hbm, vc_hbm,
                 ko_hbm, vo_hbm, pvm, slab, sem_p, sem_s):
    wid = lax.axis_index("s")
    b = jnp.where(wid < _BATCH, wid, wid - _BATCH)
    is_k = wid < _BATCH

    @pl.when(is_k)
    def _():
        pltpu.async_copy(ks_hbm.at[b, 0], slab, sem_s)

    @pl.when(jnp.logical_not(is_k))
    def _():
        pltpu.async_copy(vs_hbm.at[b, 0], slab, sem_s)

    c1 = pltpu.async_copy(si_hbm, pvm.at[pl.ds(0, 2)], sem_p)
    c2 = pltpu.async_copy(li_hbm, pvm.at[pl.ds(8, 1)], sem_p)
    c1.wait()
    c2.wait()
    v = pvm[pl.ds(0, _L)]
    begin_raw = v[0]
    end_raw = v[1]
    li_raw = v[8]
    li = jnp.clip(li_raw + jnp.where(li_raw < 0, _LAYERS, 0), 0, _LAYERS - 1)
    begin = jnp.clip(
        begin_raw + jnp.where(begin_raw < 0, _CTX, 0), 0, _CTX - 1)
    e1 = end_raw - 1
    p = jnp.clip(e1 + jnp.where(e1 < 0, _CTX, 0), 0, _CTX - 1)
    hit = p == begin

    pltpu.make_async_copy(ks_hbm.at[0, 0], slab, sem_s).wait()

    @pl.when(is_k & jnp.logical_not(hit))
    def _():
        pltpu.sync_copy(kc_hbm.at[li, b, p], slab)

    @pl.when(jnp.logical_not(is_k) & jnp.logical_not(hit))
    def _():
        pltpu.sync_copy(vc_hbm.at[li, b, p], slab)

    @pl.when(is_k)
    def _():
        pltpu.sync_copy(slab, ko_hbm.at[b, 0])

    @pl.when(jnp.logical_not(is_k))
    def _():
        pltpu.sync_copy(slab, vo_hbm.at[b, 0])


def kernel(k_state, v_state, layer_idx, slice_indices, k_cache, v_cache):
    si = slice_indices.astype(jnp.int32)
    li = jnp.asarray(layer_idx, jnp.int32).reshape(1)

    out_sds = jax.ShapeDtypeStruct(k_state.shape, k_state.dtype)
    mesh = plsc.VectorSubcoreMesh(
        core_axis_name="c", subcore_axis_name="s", num_cores=1)
    run = pl.kernel(
        _sc_kv_fetch,
        mesh=mesh,
        out_type=(out_sds, out_sds),
        scratch_types=[
            pltpu.VMEM((_L,), jnp.int32),
            pltpu.VMEM((_BATCH, 128), jnp.float32),
            pltpu.SemaphoreType.DMA,
            pltpu.SemaphoreType.DMA,
        ],
    )
    k_out, v_out = run(si, li, k_state, v_state, k_cache, v_cache)
    return (k_out, v_out)
